# baseline (speedup 1.0000x reference)
import functools

import jax
import jax.numpy as jnp
from jax import lax
from jax.experimental import pallas as pl
from jax.experimental.pallas import tpu as pltpu
from jax.experimental.pallas import tpu_sc as plsc

_NC = 2
_NS = 16
_EB = 128
_BLK = 1024
_EPS = 1e-5
_NG = 64


def _split(nbt):
    return nbt, 0


@functools.lru_cache(maxsize=None)
def _sc_deg(npad, nb0, nb1):
    rows_per_tile = npad // _NS
    nzc = rows_per_tile // _EB
    mesh = plsc.VectorSubcoreMesh(core_axis_name="c", subcore_axis_name="s",
                                  num_cores=_NC, num_subcores=_NS)

    counts = [nbc for nbc in (nb0, nb1) if nbc > 0]
    npar = len(counts)

    def body(*refs):
        dsts = refs[:npar]
        z128_hbm, o128_hbm, deg_hbm, dst_v, ones_v, acc_sh, ss = refs[npar:]
        cid = lax.axis_index("c")
        sid = lax.axis_index("s")
        tbase = sid * rows_per_tile
        pltpu.sync_copy(o128_hbm, ones_v)
        for p in range(npar):
            @pl.when(cid == p)
            def _():
                for k in range(nzc):
                    pltpu.sync_copy(z128_hbm,
                                    acc_sh.at[pl.ds(tbase + k * _EB, _EB)])
        plsc.subcore_barrier()

        for p, (nbc, dstR) in enumerate(zip(counts, dsts)):
            @pl.when(cid == p)
            def _(nbc=nbc, dstR=dstR):
                pltpu.sync_copy(dstR.at[sid], dst_v.at[pl.ds(0, nbc)])

                def fire(j, carry):
                    pltpu.async_copy(ones_v, acc_sh.at[dst_v.at[j]], ss,
                                     add=True)
                    return carry

                lax.fori_loop(0, nbc, fire, 0)

                def drain(j, carry):
                    pltpu.make_async_copy(ones_v, acc_sh.at[dst_v.at[j]],
                                          ss).wait()
                    return carry

                lax.fori_loop(0, nbc, drain, 0)

        plsc.subcore_barrier()
        for p in range(npar):
            @pl.when(cid == p)
            def _(p=p):
                pltpu.sync_copy(acc_sh.at[pl.ds(tbase, rows_per_tile)],
                                deg_hbm.at[p, pl.ds(tbase, rows_per_tile)])

    return pl.kernel(
        body,
        out_type=[jax.ShapeDtypeStruct((npar, npad, 128), jnp.float32)],
        mesh=mesh,
        scratch_types=[
            pltpu.VMEM((max(counts), _EB), jnp.int32),
            pltpu.VMEM((_EB, 128), jnp.float32),
            pltpu.VMEM_SHARED((npad, 128), jnp.float32),
            pltpu.SemaphoreType.DMA,
        ],
    )


@functools.lru_cache(maxsize=None)
def _sc_agg(c_chunks, npad, nb0, nb1):
    rows_per_tile = npad // _NS
    nzc = rows_per_tile // _EB
    counts = [nbc for nbc in (nb0, nb1) if nbc > 0]
    npar = len(counts)
    hmax = max(counts) // 2
    mesh = plsc.VectorSubcoreMesh(core_axis_name="c", subcore_axis_name="s",
                                  num_cores=_NC, num_subcores=_NS)

    def body(*refs):
        h_hbm = refs[0]
        srcs = refs[1:1 + npar]
        dsts = refs[1 + npar:1 + 2 * npar]
        (z128_hbm, parts_hbm, src_v, dst_v, rows_a, rows_b, acc_sh,
         gs_a, gs_b, ss_a, ss_b) = refs[1 + 2 * npar:]
        bufs = ((rows_a, gs_a, ss_a), (rows_b, gs_b, ss_b))
        cid = lax.axis_index("c")
        sid = lax.axis_index("s")
        tbase = sid * rows_per_tile

        for c in range(c_chunks):
            for p in range(npar):
                @pl.when(cid == p)
                def _():
                    for k in range(nzc):
                        pltpu.sync_copy(
                            z128_hbm, acc_sh.at[pl.ds(tbase + k * _EB, _EB)])
            plsc.subcore_barrier()

            for p, (nbc, srcR, dstR) in enumerate(zip(counts, srcs, dsts)):
                @pl.when(cid == p)
                def _(nbc=nbc, srcR=srcR, dstR=dstR, c=c):
                    hc = nbc // 2
                    for half in range(2):
                        pltpu.sync_copy(srcR.at[c, sid, pl.ds(half * hc, hc)],
                                        src_v.at[pl.ds(0, hc)])
                        pltpu.sync_copy(dstR.at[sid, pl.ds(half * hc, hc)],
                                        dst_v.at[pl.ds(0, hc)])
                        for b, (rv, gs, _u) in enumerate(bufs):
                            pltpu.async_copy(h_hbm.at[src_v.at[b]], rv, gs)

                        def acc_body(jj, carry):
                            for b, (rv, gs, ss) in enumerate(bufs):
                                j = jj * 2 + b
                                pltpu.make_async_copy(
                                    h_hbm.at[src_v.at[j]], rv, gs).wait()
                                pltpu.async_copy(
                                    rv, acc_sh.at[dst_v.at[j]], ss, add=True)
                                pltpu.make_async_copy(
                                    rv, acc_sh.at[dst_v.at[j]], ss).wait()

                                @pl.when(j + 2 < hc)
                                def _():
                                    pltpu.async_copy(
                                        h_hbm.at[src_v.at[j + 2]], rv, gs)
                            return carry

                        lax.fori_loop(0, hc // 2, acc_body, 0)

            plsc.subcore_barrier()
            for p in range(npar):
                @pl.when(cid == p)
                def _(p=p, c=c):
                    pltpu.sync_copy(
                        acc_sh.at[pl.ds(tbase, rows_per_tile)],
                        parts_hbm.at[c, p, pl.ds(tbase, rows_per_tile)])

    return pl.kernel(
        body,
        out_type=[jax.ShapeDtypeStruct((c_chunks, npar, npad, 128),
                                       jnp.float32)],
        mesh=mesh,
        scratch_types=[
            pltpu.VMEM((hmax, _EB), jnp.int32),
            pltpu.VMEM((hmax, _EB), jnp.int32),
            pltpu.VMEM((_EB, 128), jnp.float32),
            pltpu.VMEM((_EB, 128), jnp.float32),
            pltpu.VMEM_SHARED((npad, 128), jnp.float32),
            pltpu.SemaphoreType.DMA,
            pltpu.SemaphoreType.DMA,
            pltpu.SemaphoreType.DMA,
            pltpu.SemaphoreType.DMA,
        ],
    )


@functools.lru_cache(maxsize=None)
def _tc_layer(c_in, c_out, npad, npar):
    grid = (npad // _BLK,)

    def body(parts_ref, h_ref, deg_ref, wl_ref, wr_ref, bias_ref,
             gam_ref, bet_ref, rm_ref, rv_ref, out_ref):
        deg = sum(deg_ref[p, :, 0:1] for p in range(npar))
        inv = 1.0 / jnp.maximum(deg, 1.0)
        s = gam_ref[...] * lax.rsqrt(rv_ref[...] + _EPS)
        t = bet_ref[...] - rm_ref[...] * s
        z = jnp.zeros((_BLK, c_out * 128), jnp.float32) + bias_ref[...]
        for c in range(c_in):
            agg_c = sum(parts_ref[c, p] for p in range(npar)) * inv
            z += jnp.dot(agg_c, wl_ref[c], preferred_element_type=jnp.float32)
            z += jnp.dot(h_ref[c], wr_ref[c], preferred_element_type=jnp.float32)
        y = jnp.maximum(z * s + t, 0.0)
        for co in range(c_out):
            out_ref[co] = y[:, co * 128:(co + 1) * 128]

    d_out = c_out * 128
    return pl.pallas_call(
        body,
        grid=grid,
        in_specs=[
            pl.BlockSpec((c_in, npar, _BLK, 128), lambda i: (0, 0, i, 0)),
            pl.BlockSpec((c_in, _BLK, 128), lambda i: (0, i, 0)),
            pl.BlockSpec((npar, _BLK, 128), lambda i: (0, i, 0)),
            pl.BlockSpec((c_in, 128, d_out), lambda i: (0, 0, 0)),
            pl.BlockSpec((c_in, 128, d_out), lambda i: (0, 0, 0)),
            pl.BlockSpec((1, d_out), lambda i: (0, 0)),
            pl.BlockSpec((1, d_out), lambda i: (0, 0)),
            pl.BlockSpec((1, d_out), lambda i: (0, 0)),
            pl.BlockSpec((1, d_out), lambda i: (0, 0)),
            pl.BlockSpec((1, d_out), lambda i: (0, 0)),
        ],
        out_specs=pl.BlockSpec((c_out, _BLK, 128), lambda i: (0, i, 0)),
        out_shape=jax.ShapeDtypeStruct((c_out, npad, 128), jnp.float32),
    )


@functools.lru_cache(maxsize=None)
def _pool_fc(cd, npad, num_classes):
    grid = (npad // _BLK,)
    nsteps = grid[0]

    def body(h_ref, batch_ref, fcw_ref, fcb_ref, out_ref, acc_ref, cnt_ref):
        i = pl.program_id(0)

        @pl.when(i == 0)
        def _():
            acc_ref[...] = jnp.zeros_like(acc_ref)
            cnt_ref[...] = jnp.zeros_like(cnt_ref)

        gids = lax.broadcasted_iota(jnp.int32, (_NG, _BLK), 0)
        p = (gids == batch_ref[...]).astype(jnp.float32)
        for c in range(cd):
            acc_ref[:, c * 128:(c + 1) * 128] += jnp.dot(
                p, h_ref[c], preferred_element_type=jnp.float32)
        cnt_ref[...] += jnp.sum(p, axis=1, keepdims=True)

        @pl.when(i == nsteps - 1)
        def _():
            pooled = acc_ref[...] / jnp.maximum(cnt_ref[...], 1.0)
            o = jnp.zeros((_NG, num_classes), jnp.float32) + fcb_ref[...]
            for c in range(cd):
                o += jnp.dot(pooled[:, c * 128:(c + 1) * 128], fcw_ref[c],
                             preferred_element_type=jnp.float32)
            out_ref[...] = o

    return pl.pallas_call(
        body,
        grid=grid,
        in_specs=[
            pl.BlockSpec((cd, _BLK, 128), lambda i: (0, i, 0)),
            pl.BlockSpec((1, _BLK), lambda i: (0, i)),
            pl.BlockSpec((cd, 128, num_classes), lambda i: (0, 0, 0)),
            pl.BlockSpec((1, num_classes), lambda i: (0, 0)),
        ],
        out_specs=pl.BlockSpec((_NG, num_classes), lambda i: (0, 0)),
        out_shape=jax.ShapeDtypeStruct((_NG, num_classes), jnp.float32),
        scratch_shapes=[pltpu.VMEM((_NG, cd * 128), jnp.float32),
                        pltpu.VMEM((_NG, 1), jnp.float32)],
    )


def kernel(x, edge_index, batch, params):
    n, d_in = x.shape
    e = edge_index.shape[1]
    npad = -(-n // _BLK) * _BLK
    ebatch = _NS * _EB * 16
    epad = -(-e // ebatch) * ebatch
    nbt = epad // (_NS * _EB)
    nb0, nb1 = _split(nbt)
    pad_e = epad - e

    src_p = jnp.concatenate([edge_index[0], jnp.zeros((pad_e,), jnp.int32)])
    dst_p = jnp.concatenate([edge_index[1], jnp.full((pad_e,), n, jnp.int32)])
    cut = _NS * nb0 * _EB
    srcs = [src_p[:cut].reshape(_NS, nb0, _EB)]
    dsts = [dst_p[:cut].reshape(_NS, nb0, _EB)]
    if nb1 > 0:
        srcs.append(src_p[cut:].reshape(_NS, nb1, _EB))
        dsts.append(dst_p[cut:].reshape(_NS, nb1, _EB))
    npar = len(srcs)

    src_idx = {}
    for c_chunks in sorted({d_in // 128, 4}):
        off = (jnp.arange(c_chunks, dtype=jnp.int32) * npad)[:, None, None, None]
        src_idx[c_chunks] = [s[None] + off for s in srcs]

    z128 = jnp.zeros((_EB, 128), jnp.float32)
    o128 = jnp.ones((_EB, 128), jnp.float32)

    x_pad = jnp.pad(x, ((0, npad - n), (0, 0)))
    h = x_pad.reshape(npad, d_in // 128, 128).transpose(1, 0, 2)

    (deg,) = _sc_deg(npad, nb0, nb1)(*dsts, z128, o128)

    for layer in params["layers"]:
        c_in = h.shape[0]
        c_out = layer["lin_l_w"].shape[1] // 128
        h_flat = h.reshape(c_in * npad, 128)
        (parts,) = _sc_agg(c_in, npad, nb0, nb1)(
            h_flat, *src_idx[c_in], *dsts, z128)
        tc = _tc_layer(c_in, c_out, npad, npar)
        d_out = c_out * 128
        h = tc(parts, h, deg,
               layer["lin_l_w"].reshape(c_in, 128, d_out),
               layer["lin_r_w"].reshape(c_in, 128, d_out),
               layer["lin_l_b"][None], layer["bn_gamma"][None],
               layer["bn_beta"][None], layer["bn_rm"][None],
               layer["bn_rv"][None])

    batch_p = jnp.concatenate(
        [batch, jnp.full((npad - n,), _NG, jnp.int32)]).reshape(1, npad)
    num_classes = params["fc_b"].shape[0]
    pool = _pool_fc(h.shape[0], npad, num_classes)
    return pool(h, batch_p,
                params["fc_w"].reshape(h.shape[0], 128, num_classes),
                params["fc_b"][None])


# speedup vs baseline: 1.2507x; 1.2507x over previous
import functools

import jax
import jax.numpy as jnp
from jax import lax
from jax.experimental import pallas as pl
from jax.experimental.pallas import tpu as pltpu
from jax.experimental.pallas import tpu_sc as plsc

_NC = 2
_NS = 16
_EB = 128
_BLK = 1024
_EPS = 1e-5
_NG = 64


def _split(nbt):
    nb0 = min(nbt - 16, max(16, nbt * 4 // 5 // 16 * 16))
    return nb0, nbt - nb0


@functools.lru_cache(maxsize=None)
def _sc_deg(npad, nb0, nb1):
    rows_per_tile = npad // _NS
    nzc = rows_per_tile // _EB
    mesh = plsc.VectorSubcoreMesh(core_axis_name="c", subcore_axis_name="s",
                                  num_cores=_NC, num_subcores=_NS)

    def body(dst0_hbm, dst1_hbm, z16_hbm, o16_hbm, deg_hbm,
             dst_v, ones_v, zero_v, acc_sh, ss):
        cid = lax.axis_index("c")
        sid = lax.axis_index("s")
        tbase = sid * rows_per_tile
        pltpu.sync_copy(o16_hbm, ones_v)
        pltpu.sync_copy(z16_hbm, zero_v)
        for k in range(nzc):
            pltpu.sync_copy(zero_v, acc_sh.at[pl.ds(tbase + k * _EB, _EB)])
        plsc.subcore_barrier()

        for which, nbc, dstR in ((0, nb0, dst0_hbm), (1, nb1, dst1_hbm)):
            @pl.when(cid == which)
            def _(nbc=nbc, dstR=dstR):
                pltpu.sync_copy(dstR.at[sid], dst_v.at[pl.ds(0, nbc)])

                def fire(j, carry):
                    pltpu.async_copy(ones_v, acc_sh.at[dst_v.at[j]], ss,
                                     add=True)
                    return carry

                lax.fori_loop(0, nbc, fire, 0)

                def drain(j, carry):
                    pltpu.make_async_copy(ones_v, acc_sh.at[dst_v.at[j]],
                                          ss).wait()
                    return carry

                lax.fori_loop(0, nbc, drain, 0)

        plsc.subcore_barrier()
        pltpu.sync_copy(acc_sh.at[pl.ds(tbase, rows_per_tile)],
                        deg_hbm.at[cid, pl.ds(tbase, rows_per_tile)])

    return pl.kernel(
        body,
        out_type=[jax.ShapeDtypeStruct((_NC, npad, 128), jnp.float32)],
        mesh=mesh,
        scratch_types=[
            pltpu.VMEM((max(nb0, nb1), _EB), jnp.int32),
            pltpu.VMEM((_EB, 128), jnp.float32),
            pltpu.VMEM((_EB, 128), jnp.float32),
            pltpu.VMEM_SHARED((npad, 128), jnp.float32),
            pltpu.SemaphoreType.DMA,
        ],
    )


@functools.lru_cache(maxsize=None)
def _sc_agg(c_chunks, npad, nb0, nb1):
    rows_per_tile = npad // _NS
    nzc = rows_per_tile // _EB
    hmax = max(nb0, nb1) // 2
    mesh = plsc.VectorSubcoreMesh(core_axis_name="c", subcore_axis_name="s",
                                  num_cores=_NC, num_subcores=_NS)

    def body(h_hbm, src0_hbm, src1_hbm, dst0_hbm, dst1_hbm, z128_hbm,
             parts_hbm, src_v, dst_v, rows_a, rows_b, acc_sh,
             gs_a, gs_b, ss_a, ss_b):
        bufs = ((rows_a, gs_a, ss_a), (rows_b, gs_b, ss_b))
        cid = lax.axis_index("c")
        sid = lax.axis_index("s")
        tbase = sid * rows_per_tile

        for c in range(c_chunks):
            pltpu.sync_copy(z128_hbm, rows_a)
            for k in range(nzc):
                pltpu.sync_copy(rows_a, acc_sh.at[pl.ds(tbase + k * _EB, _EB)])
            plsc.subcore_barrier()

            for which, nbc, srcR, dstR in ((0, nb0, src0_hbm, dst0_hbm),
                                           (1, nb1, src1_hbm, dst1_hbm)):
                @pl.when(cid == which)
                def _(nbc=nbc, srcR=srcR, dstR=dstR, c=c):
                    hc = nbc // 2
                    for half in range(2):
                        pltpu.sync_copy(srcR.at[c, sid, pl.ds(half * hc, hc)],
                                        src_v.at[pl.ds(0, hc)])
                        pltpu.sync_copy(dstR.at[sid, pl.ds(half * hc, hc)],
                                        dst_v.at[pl.ds(0, hc)])
                        for b, (rv, gs, _u) in enumerate(bufs):
                            pltpu.async_copy(h_hbm.at[src_v.at[b]], rv, gs)

                        def acc_body(jj, carry):
                            for b, (rv, gs, ss) in enumerate(bufs):
                                j = jj * 2 + b
                                pltpu.make_async_copy(
                                    h_hbm.at[src_v.at[j]], rv, gs).wait()
                                pltpu.async_copy(
                                    rv, acc_sh.at[dst_v.at[j]], ss, add=True)
                                pltpu.make_async_copy(
                                    rv, acc_sh.at[dst_v.at[j]], ss).wait()

                                @pl.when(j + 2 < hc)
                                def _():
                                    pltpu.async_copy(
                                        h_hbm.at[src_v.at[j + 2]], rv, gs)
                            return carry

                        lax.fori_loop(0, hc // 2, acc_body, 0)

            plsc.subcore_barrier()
            pltpu.sync_copy(acc_sh.at[pl.ds(tbase, rows_per_tile)],
                            parts_hbm.at[c, cid, pl.ds(tbase, rows_per_tile)])

    return pl.kernel(
        body,
        out_type=[jax.ShapeDtypeStruct((c_chunks, _NC, npad, 128),
                                       jnp.float32)],
        mesh=mesh,
        scratch_types=[
            pltpu.VMEM((hmax, _EB), jnp.int32),
            pltpu.VMEM((hmax, _EB), jnp.int32),
            pltpu.VMEM((_EB, 128), jnp.float32),
            pltpu.VMEM((_EB, 128), jnp.float32),
            pltpu.VMEM_SHARED((npad, 128), jnp.float32),
            pltpu.SemaphoreType.DMA,
            pltpu.SemaphoreType.DMA,
            pltpu.SemaphoreType.DMA,
            pltpu.SemaphoreType.DMA,
        ],
    )


@functools.lru_cache(maxsize=None)
def _tc_layer(c_in, c_out, npad):
    grid = (npad // _BLK,)

    def body(parts_ref, h_ref, deg_ref, wl_ref, wr_ref, bias_ref,
             gam_ref, bet_ref, rm_ref, rv_ref, out_ref):
        deg = deg_ref[0, :, 0:1] + deg_ref[1, :, 0:1]
        inv = 1.0 / jnp.maximum(deg, 1.0)
        s = gam_ref[...] * lax.rsqrt(rv_ref[...] + _EPS)
        t = bet_ref[...] - rm_ref[...] * s
        z = jnp.zeros((_BLK, c_out * 128), jnp.float32) + bias_ref[...]
        for c in range(c_in):
            agg_c = (parts_ref[c, 0] + parts_ref[c, 1]) * inv
            z += jnp.dot(agg_c, wl_ref[c], preferred_element_type=jnp.float32)
            z += jnp.dot(h_ref[c], wr_ref[c], preferred_element_type=jnp.float32)
        y = jnp.maximum(z * s + t, 0.0)
        for co in range(c_out):
            out_ref[co] = y[:, co * 128:(co + 1) * 128]

    d_out = c_out * 128
    return pl.pallas_call(
        body,
        grid=grid,
        in_specs=[
            pl.BlockSpec((c_in, _NC, _BLK, 128), lambda i: (0, 0, i, 0)),
            pl.BlockSpec((c_in, _BLK, 128), lambda i: (0, i, 0)),
            pl.BlockSpec((_NC, _BLK, 128), lambda i: (0, i, 0)),
            pl.BlockSpec((c_in, 128, d_out), lambda i: (0, 0, 0)),
            pl.BlockSpec((c_in, 128, d_out), lambda i: (0, 0, 0)),
            pl.BlockSpec((1, d_out), lambda i: (0, 0)),
            pl.BlockSpec((1, d_out), lambda i: (0, 0)),
            pl.BlockSpec((1, d_out), lambda i: (0, 0)),
            pl.BlockSpec((1, d_out), lambda i: (0, 0)),
            pl.BlockSpec((1, d_out), lambda i: (0, 0)),
        ],
        out_specs=pl.BlockSpec((c_out, _BLK, 128), lambda i: (0, i, 0)),
        out_shape=jax.ShapeDtypeStruct((c_out, npad, 128), jnp.float32),
    )


@functools.lru_cache(maxsize=None)
def _pool_fc(cd, npad, num_classes):
    grid = (npad // _BLK,)
    nsteps = grid[0]

    def body(h_ref, batch_ref, fcw_ref, fcb_ref, out_ref, acc_ref, cnt_ref):
        i = pl.program_id(0)

        @pl.when(i == 0)
        def _():
            acc_ref[...] = jnp.zeros_like(acc_ref)
            cnt_ref[...] = jnp.zeros_like(cnt_ref)

        gids = lax.broadcasted_iota(jnp.int32, (_NG, _BLK), 0)
        p = (gids == batch_ref[...]).astype(jnp.float32)
        for c in range(cd):
            acc_ref[:, c * 128:(c + 1) * 128] += jnp.dot(
                p, h_ref[c], preferred_element_type=jnp.float32)
        cnt_ref[...] += jnp.sum(p, axis=1, keepdims=True)

        @pl.when(i == nsteps - 1)
        def _():
            pooled = acc_ref[...] / jnp.maximum(cnt_ref[...], 1.0)
            o = jnp.zeros((_NG, num_classes), jnp.float32) + fcb_ref[...]
            for c in range(cd):
                o += jnp.dot(pooled[:, c * 128:(c + 1) * 128], fcw_ref[c],
                             preferred_element_type=jnp.float32)
            out_ref[...] = o

    return pl.pallas_call(
        body,
        grid=grid,
        in_specs=[
            pl.BlockSpec((cd, _BLK, 128), lambda i: (0, i, 0)),
            pl.BlockSpec((1, _BLK), lambda i: (0, i)),
            pl.BlockSpec((cd, 128, num_classes), lambda i: (0, 0, 0)),
            pl.BlockSpec((1, num_classes), lambda i: (0, 0)),
        ],
        out_specs=pl.BlockSpec((_NG, num_classes), lambda i: (0, 0)),
        out_shape=jax.ShapeDtypeStruct((_NG, num_classes), jnp.float32),
        scratch_shapes=[pltpu.VMEM((_NG, cd * 128), jnp.float32),
                        pltpu.VMEM((_NG, 1), jnp.float32)],
    )


def kernel(x, edge_index, batch, params):
    n, d_in = x.shape
    e = edge_index.shape[1]
    npad = -(-n // _BLK) * _BLK
    ebatch = _NS * _EB * 16
    epad = -(-e // ebatch) * ebatch
    nbt = epad // (_NS * _EB)
    nb0, nb1 = _split(nbt)
    pad_e = epad - e

    src_p = jnp.concatenate([edge_index[0], jnp.zeros((pad_e,), jnp.int32)])
    dst_p = jnp.concatenate([edge_index[1], jnp.full((pad_e,), n, jnp.int32)])
    cut = _NS * nb0 * _EB
    src0 = src_p[:cut].reshape(_NS, nb0, _EB)
    src1 = src_p[cut:].reshape(_NS, nb1, _EB)
    dst0 = dst_p[:cut].reshape(_NS, nb0, _EB)
    dst1 = dst_p[cut:].reshape(_NS, nb1, _EB)

    src_idx = {}
    for c_chunks in sorted({d_in // 128, 4}):
        off = (jnp.arange(c_chunks, dtype=jnp.int32) * npad)[:, None, None, None]
        src_idx[c_chunks] = (src0[None] + off, src1[None] + off)

    z128 = jnp.zeros((_EB, 128), jnp.float32)
    o128 = jnp.ones((_EB, 128), jnp.float32)

    x_pad = jnp.pad(x, ((0, npad - n), (0, 0)))
    h = x_pad.reshape(npad, d_in // 128, 128).transpose(1, 0, 2)

    (deg,) = _sc_deg(npad, nb0, nb1)(dst0, dst1, z128, o128)

    for layer in params["layers"]:
        c_in = h.shape[0]
        c_out = layer["lin_l_w"].shape[1] // 128
        h_flat = h.reshape(c_in * npad, 128)
        s0, s1 = src_idx[c_in]
        (parts,) = _sc_agg(c_in, npad, nb0, nb1)(
            h_flat, s0, s1, dst0, dst1, z128)
        tc = _tc_layer(c_in, c_out, npad)
        d_out = c_out * 128
        h = tc(parts, h, deg,
               layer["lin_l_w"].reshape(c_in, 128, d_out),
               layer["lin_r_w"].reshape(c_in, 128, d_out),
               layer["lin_l_b"][None], layer["bn_gamma"][None],
               layer["bn_beta"][None], layer["bn_rm"][None],
               layer["bn_rv"][None])

    batch_p = jnp.concatenate(
        [batch, jnp.full((npad - n,), _NG, jnp.int32)]).reshape(1, npad)
    num_classes = params["fc_b"].shape[0]
    pool = _pool_fc(h.shape[0], npad, num_classes)
    return pool(h, batch_p,
                params["fc_w"].reshape(h.shape[0], 128, num_classes),
                params["fc_b"][None])
